# baseline (speedup 1.0000x reference)
import functools

import jax
import jax.numpy as jnp
from jax import lax
from jax.experimental import pallas as pl
from jax.experimental.pallas import tpu as pltpu
from jax.experimental.pallas import tpu_sc as plsc

_NC, _NS = 2, 16
_NW = _NC * _NS


def kernel(inputs, table):
    del inputs
    rows, dim = table.shape
    rows_per_w = rows // _NW
    mesh = plsc.VectorSubcoreMesh(core_axis_name="c", subcore_axis_name="s")

    @functools.partial(
        pl.kernel,
        mesh=mesh,
        out_type=jax.ShapeDtypeStruct((rows, dim), table.dtype),
    )
    def _sc_copy(table_hbm, out_hbm):
        wid = lax.axis_index("s") * _NC + lax.axis_index("c")
        base = wid * rows_per_w
        pltpu.sync_copy(
            table_hbm.at[pl.ds(base, rows_per_w), :],
            out_hbm.at[pl.ds(base, rows_per_w), :],
        )

    return _sc_copy(table)


# speedup vs baseline: 24.8251x; 24.8251x over previous
import functools

import jax
import jax.numpy as jnp
from jax import lax
from jax.experimental import pallas as pl
from jax.experimental.pallas import tpu as pltpu
from jax.experimental.pallas import tpu_sc as plsc

_NC, _NS = 2, 16
_NW = _NC * _NS
_CHUNK = 32
_NBUF = 3


def kernel(inputs, table):
    del inputs
    rows, dim = table.shape
    rows_per_w = rows // _NW
    nchunks = rows_per_w // _CHUNK
    mesh = plsc.VectorSubcoreMesh(core_axis_name="c", subcore_axis_name="s")

    @functools.partial(
        pl.kernel,
        mesh=mesh,
        out_type=jax.ShapeDtypeStruct((rows, dim), table.dtype),
        scratch_types=[
            pltpu.VMEM((_NBUF, _CHUNK, dim), table.dtype),
            pltpu.SemaphoreType.DMA,
            pltpu.SemaphoreType.DMA,
        ],
    )
    def _sc_copy(table_hbm, out_hbm, buf, rsem, wsem):
        wid = lax.axis_index("s") * _NC + lax.axis_index("c")
        base = wid * rows_per_w

        def read(c):
            return pltpu.async_copy(
                table_hbm.at[pl.ds(base + c * _CHUNK, _CHUNK), :],
                buf.at[c % _NBUF],
                rsem,
            )

        def write(c):
            return pltpu.async_copy(
                buf.at[c % _NBUF],
                out_hbm.at[pl.ds(base + c * _CHUNK, _CHUNK), :],
                wsem,
            )

        reads = [None] * nchunks
        writes = [None] * nchunks
        for c in range(min(_NBUF, nchunks)):
            reads[c] = read(c)
        for c in range(nchunks):
            reads[c].wait()
            writes[c] = write(c)
            nxt = c + _NBUF
            if nxt < nchunks:
                writes[c].wait()
                reads[nxt] = read(nxt)
        for c in range(max(0, nchunks - _NBUF), nchunks):
            writes[c].wait()

    return _sc_copy(table)
